# baseline (speedup 1.0000x reference)
import functools

import jax
import jax.numpy as jnp
from jax import lax
from jax.experimental import pallas as pl
from jax.experimental.pallas import tpu as pltpu
from jax.experimental.pallas import tpu_sc as plsc

N = 10000
NP = 10240
E = 160000
NC, NS, LANES = 2, 16, 16
ROWS_PER_TILE = NP // NS
EDGES_PER_TILE_AB = E // NS
EDGES_PER_TILE_C = E // (NC * NS)
K = 80
KE = 128
NF_E = EDGES_PER_TILE_AB // KE
REM_E = EDGES_PER_TILE_AB - NF_E * KE
ZROWS = 64

_mesh = functools.partial(
    plsc.VectorSubcoreMesh, core_axis_name="c", subcore_axis_name="s")


def _zero_fill(buf, ncols):
    def row(i, _):
        for j in range(ncols // LANES):
            buf[i, pl.ds(j * LANES, LANES)] = jnp.zeros((LANES,), jnp.float32)
        return 0
    lax.fori_loop(0, ZROWS, row, 0)


def _zero_shared(zbuf, shared, row0, chunk_idx=None):
    for b in range(ROWS_PER_TILE // ZROWS):
        dst = shared.at[pl.ds(row0 + b * ZROWS, ZROWS)]
        pltpu.sync_copy(zbuf, dst)


def _edge_loop(table, src_hbm, dst_hbm, acc_sh, sidx, didx, rows, gsem,
               ssem, sidx_t, didx_t, tile_base):
    def load_idx(g, b):
        base = tile_base + g * KE
        pltpu.sync_copy(src_hbm.at[pl.ds(base, KE)], sidx[b])
        pltpu.sync_copy(dst_hbm.at[pl.ds(base, KE)], didx[b])

    def gather_start(b):
        pltpu.async_copy(table.at[sidx[b]], rows[b], gsem[b])

    def gather_wait(b):
        pltpu.make_async_copy(table.at[sidx[b]], rows[b], gsem[b]).wait()

    def scatter_start(b):
        pltpu.async_copy(rows[b], acc_sh.at[didx[b]], ssem[b], add=True)

    def scatter_wait(b):
        pltpu.make_async_copy(rows[b], acc_sh.at[didx[b]], ssem[b]).wait()

    def body(g, b):
        scatter_wait(b)
        load_idx(g, b)
        gather_start(b)
        gather_wait(1 - b)
        scatter_start(1 - b)

    load_idx(0, 0)
    gather_start(0)
    load_idx(1, 1)
    gather_start(1)
    gather_wait(0)
    scatter_start(0)

    @pl.loop(2, NF_E, step=2)
    def _(g0):
        body(g0, 0)
        body(g0 + 1, 1)

    base_t = tile_base + NF_E * KE
    rows_t = rows[0].at[pl.ds(0, REM_E)]
    scatter_wait(0)
    pltpu.sync_copy(src_hbm.at[pl.ds(base_t, REM_E)], sidx_t)
    pltpu.sync_copy(dst_hbm.at[pl.ds(base_t, REM_E)], didx_t)
    pltpu.async_copy(table.at[sidx_t], rows_t, gsem[0])
    gather_wait(1)
    scatter_start(1)
    pltpu.make_async_copy(table.at[sidx_t], rows_t, gsem[0]).wait()
    pltpu.async_copy(rows_t, acc_sh.at[didx_t], ssem[0], add=True)
    scatter_wait(1)
    pltpu.make_async_copy(rows_t, acc_sh.at[didx_t], ssem[0]).wait()


KD = 128
NF_D = EDGES_PER_TILE_C // KD
REM_D = EDGES_PER_TILE_C - NF_D * KD


def _deg_body(dst_hbm, deg_out, deg_sh, ones_v, didx0, didx1,
              ssem0, ssem1, didx_t):
    c = lax.axis_index("c")
    s = lax.axis_index("s")
    row0 = s * ROWS_PER_TILE
    wid = s * NC + c
    base0 = wid * EDGES_PER_TILE_C
    didx = (didx0, didx1)
    ssem = (ssem0, ssem1)

    _zero_fill(ones_v, 128)
    _zero_shared(ones_v.at[pl.ds(0, ZROWS)], deg_sh, row0)

    plsc.subcore_barrier()

    def orow(i, _):
        for j in range(128 // LANES):
            ones_v[i, pl.ds(j * LANES, LANES)] = jnp.ones((LANES,), jnp.float32)
        return 0
    lax.fori_loop(0, KD, orow, 0)

    def load_idx(g, b):
        pltpu.sync_copy(dst_hbm.at[pl.ds(base0 + g * KD, KD)], didx[b])

    def scatter_start(b):
        pltpu.async_copy(ones_v, deg_sh.at[didx[b]], ssem[b], add=True)

    def scatter_wait(b):
        pltpu.make_async_copy(ones_v, deg_sh.at[didx[b]], ssem[b]).wait()

    def body(g, b):
        scatter_wait(b)
        load_idx(g, b)
        scatter_start(b)

    load_idx(0, 0)
    scatter_start(0)
    load_idx(1, 1)
    scatter_start(1)

    @pl.loop(2, NF_D - 1, step=2)
    def _(g0):
        body(g0, 0)
        body(g0 + 1, 1)

    body(NF_D - 1, 0)

    ones_t = ones_v.at[pl.ds(0, REM_D)]
    scatter_wait(1)
    pltpu.sync_copy(dst_hbm.at[pl.ds(base0 + NF_D * KD, REM_D)], didx_t)
    pltpu.async_copy(ones_t, deg_sh.at[didx_t], ssem[1], add=True)
    scatter_wait(0)
    pltpu.make_async_copy(ones_t, deg_sh.at[didx_t], ssem[1]).wait()

    plsc.subcore_barrier()

    my_rows = pl.ds(row0, ROWS_PER_TILE)

    @pl.when(c == 0)
    def _():
        pltpu.sync_copy(deg_sh.at[my_rows], deg_out.at[0, my_rows])

    @pl.when(c == 1)
    def _():
        pltpu.sync_copy(deg_sh.at[my_rows], deg_out.at[1, my_rows])


def _deg(dst):
    f = pl.kernel(
        _deg_body,
        out_type=jax.ShapeDtypeStruct((2, NP, 128), jnp.float32),
        mesh=_mesh(),
        scratch_types=[
            pltpu.VMEM_SHARED((NP, 128), jnp.float32),
            pltpu.VMEM((KD, 128), jnp.float32),
            pltpu.VMEM((KD,), jnp.int32),
            pltpu.VMEM((KD,), jnp.int32),
            pltpu.SemaphoreType.DMA,
            pltpu.SemaphoreType.DMA,
            pltpu.VMEM((REM_D,), jnp.int32),
        ],
    )
    return f(dst)


def _seg_a_body(x0, x1, src_hbm, dst_hbm, agg_out,
                acc_sh, zbuf, sidx0, sidx1, didx0, didx1, rows0, rows1,
                gsem0, gsem1, ssem0, ssem1, sidx_t, didx_t):
    c = lax.axis_index("c")
    s = lax.axis_index("s")
    row0 = s * ROWS_PER_TILE
    tile_base = s * EDGES_PER_TILE_AB
    sidx = (sidx0, sidx1)
    didx = (didx0, didx1)
    rows = (rows0, rows1)
    gsem = (gsem0, gsem1)
    ssem = (ssem0, ssem1)

    _zero_fill(zbuf, 128)
    _zero_shared(zbuf, acc_sh, row0)

    plsc.subcore_barrier()

    @pl.when(c == 0)
    def _():
        _edge_loop(x0, src_hbm, dst_hbm, acc_sh, sidx, didx, rows, gsem,
                   ssem, sidx_t, didx_t, tile_base)

    @pl.when(c == 1)
    def _():
        _edge_loop(x1, src_hbm, dst_hbm, acc_sh, sidx, didx, rows, gsem,
                   ssem, sidx_t, didx_t, tile_base)

    plsc.subcore_barrier()

    my_rows = pl.ds(row0, ROWS_PER_TILE)

    @pl.when(c == 0)
    def _():
        pltpu.sync_copy(acc_sh.at[my_rows], agg_out.at[0, my_rows])

    @pl.when(c == 1)
    def _():
        pltpu.sync_copy(acc_sh.at[my_rows], agg_out.at[1, my_rows])


def _seg_a(x0, x1, src, dst):
    f = pl.kernel(
        _seg_a_body,
        out_type=jax.ShapeDtypeStruct((2, NP, 128), jnp.float32),
        mesh=_mesh(),
        scratch_types=[
            pltpu.VMEM_SHARED((NP, 128), jnp.float32),
            pltpu.VMEM((ZROWS, 128), jnp.float32),
            pltpu.VMEM((KE,), jnp.int32),
            pltpu.VMEM((KE,), jnp.int32),
            pltpu.VMEM((KE,), jnp.int32),
            pltpu.VMEM((KE,), jnp.int32),
            pltpu.VMEM((KE, 128), jnp.float32),
            pltpu.VMEM((KE, 128), jnp.float32),
            pltpu.SemaphoreType.DMA,
            pltpu.SemaphoreType.DMA,
            pltpu.SemaphoreType.DMA,
            pltpu.SemaphoreType.DMA,
            pltpu.VMEM((REM_E,), jnp.int32),
            pltpu.VMEM((REM_E,), jnp.int32),
        ],
    )
    return f(x0, x1, src, dst)


def _seg_b_body(h0, h1, h2, h3, src_hbm, dst_hbm, agg_out,
                acc_sh, zbuf, sidx0, sidx1, didx0, didx1, rows0, rows1,
                gsem0, gsem1, ssem0, ssem1, sidx_t, didx_t):
    c = lax.axis_index("c")
    s = lax.axis_index("s")
    row0 = s * ROWS_PER_TILE
    tile_base = s * EDGES_PER_TILE_AB
    my_rows = pl.ds(row0, ROWS_PER_TILE)
    sidx = (sidx0, sidx1)
    didx = (didx0, didx1)
    rows = (rows0, rows1)
    gsem = (gsem0, gsem1)
    ssem = (ssem0, ssem1)

    _zero_fill(zbuf, 128)

    def do_chunk(table, out_idx):
        _zero_shared(zbuf, acc_sh, row0)
        plsc.subcore_barrier()
        _edge_loop(table, src_hbm, dst_hbm, acc_sh, sidx, didx, rows, gsem,
                   ssem, sidx_t, didx_t, tile_base)
        plsc.subcore_barrier()
        pltpu.sync_copy(acc_sh.at[my_rows], agg_out.at[out_idx, my_rows])

    @pl.when(c == 0)
    def _():
        do_chunk(h0, 0)
        do_chunk(h1, 1)

    @pl.when(c == 1)
    def _():
        do_chunk(h2, 2)
        do_chunk(h3, 3)


def _seg_b(h0, h1, h2, h3, src, dst):
    f = pl.kernel(
        _seg_b_body,
        out_type=jax.ShapeDtypeStruct((4, NP, 128), jnp.float32),
        mesh=_mesh(),
        scratch_types=[
            pltpu.VMEM_SHARED((NP, 128), jnp.float32),
            pltpu.VMEM((ZROWS, 128), jnp.float32),
            pltpu.VMEM((KE,), jnp.int32),
            pltpu.VMEM((KE,), jnp.int32),
            pltpu.VMEM((KE,), jnp.int32),
            pltpu.VMEM((KE,), jnp.int32),
            pltpu.VMEM((KE, 128), jnp.float32),
            pltpu.VMEM((KE, 128), jnp.float32),
            pltpu.SemaphoreType.DMA,
            pltpu.SemaphoreType.DMA,
            pltpu.SemaphoreType.DMA,
            pltpu.SemaphoreType.DMA,
            pltpu.VMEM((REM_E,), jnp.int32),
            pltpu.VMEM((REM_E,), jnp.int32),
        ],
    )
    return f(h0, h1, h2, h3, src, dst)


def _score_body(s_tbl, t_tbl, src_hbm, dst_hbm, out,
                sidx0, sidx1, didx0, didx1, ba0, ba1, bb0, bb1,
                gsa0, gsa1, gsb0, gsb1, wsem0, wsem1):
    c = lax.axis_index("c")
    s = lax.axis_index("s")
    wid = s * NC + c
    base0 = wid * EDGES_PER_TILE_C
    sidx = (sidx0, sidx1)
    didx = (didx0, didx1)
    ba = (ba0, ba1)
    bb = (bb0, bb1)
    gsa = (gsa0, gsa1)
    gsb = (gsb0, gsb1)
    wsem = (wsem0, wsem1)

    NF = EDGES_PER_TILE_C // K
    REM = EDGES_PER_TILE_C - NF * K

    def load_and_gather(g, b, n):
        base = base0 + g * K
        pltpu.sync_copy(src_hbm.at[pl.ds(base, n)], sidx[b].at[pl.ds(0, n)])
        pltpu.sync_copy(dst_hbm.at[pl.ds(base, n)], didx[b].at[pl.ds(0, n)])
        pltpu.async_copy(s_tbl.at[sidx[b].at[pl.ds(0, n)]],
                         ba[b].at[pl.ds(0, n)], gsa[b])
        pltpu.async_copy(t_tbl.at[didx[b].at[pl.ds(0, n)]],
                         bb[b].at[pl.ds(0, n)], gsb[b])

    def process(g, b, n):
        pltpu.make_async_copy(s_tbl.at[sidx[b].at[pl.ds(0, n)]],
                              ba[b].at[pl.ds(0, n)], gsa[b]).wait()
        pltpu.make_async_copy(t_tbl.at[didx[b].at[pl.ds(0, n)]],
                              bb[b].at[pl.ds(0, n)], gsb[b]).wait()

        @plsc.parallel_loop(0, n, unroll=4)
        def _(i):
            for j in range(256 // LANES):
                sl = pl.ds(j * LANES, LANES)
                plsc.addupdate(ba[b].at[i, sl], bb[b][i, sl])
        pltpu.async_copy(ba[b].at[pl.ds(0, n)],
                         out.at[pl.ds(base0 + g * K, n)], wsem[b])

    def write_wait(g, b, n):
        pltpu.make_async_copy(ba[b].at[pl.ds(0, n)],
                              out.at[pl.ds(base0 + g * K, n)], wsem[b]).wait()

    def body(g, b):
        write_wait(g - 2, b, K)
        load_and_gather(g, b, K)
        process(g - 1, 1 - b, K)

    load_and_gather(0, 0, K)
    load_and_gather(1, 1, K)
    process(0, 0, K)

    @pl.loop(2, NF, step=2)
    def _(g0):
        body(g0, 0)
        body(g0 + 1, 1)

    write_wait(NF - 2, 0, K)
    load_and_gather(NF, 0, REM)
    process(NF - 1, 1, K)
    write_wait(NF - 1, 1, K)
    process(NF, 0, REM)
    write_wait(NF, 0, REM)


def _score(s_tbl, t_tbl, src, dst):
    f = pl.kernel(
        _score_body,
        out_type=jax.ShapeDtypeStruct((E, 256), jnp.float32),
        mesh=_mesh(),
        scratch_types=[
            pltpu.VMEM((K,), jnp.int32),
            pltpu.VMEM((K,), jnp.int32),
            pltpu.VMEM((K,), jnp.int32),
            pltpu.VMEM((K,), jnp.int32),
            pltpu.VMEM((K, 256), jnp.float32),
            pltpu.VMEM((K, 256), jnp.float32),
            pltpu.VMEM((K, 256), jnp.float32),
            pltpu.VMEM((K, 256), jnp.float32),
            pltpu.SemaphoreType.DMA,
            pltpu.SemaphoreType.DMA,
            pltpu.SemaphoreType.DMA,
            pltpu.SemaphoreType.DMA,
            pltpu.SemaphoreType.DMA,
            pltpu.SemaphoreType.DMA,
        ],
    )
    return f(s_tbl, t_tbl, src, dst)


_RB = 2048


def _tc1(x, agg, deg, Ws1, Wn1, b1):
    def body(x_ref, agg_ref, deg_ref, ws_ref, wn_ref, b_ref, o0, o1, o2, o3):
        deg = deg_ref[0, :, 0:1] + deg_ref[1, :, 0:1]
        rdeg = 1.0 / jnp.maximum(deg, 1.0)
        h = jnp.dot(x_ref[...], ws_ref[...],
                    preferred_element_type=jnp.float32)
        h += jnp.dot(agg_ref[0] * rdeg, wn_ref[0:128, :],
                     preferred_element_type=jnp.float32)
        h += jnp.dot(agg_ref[1] * rdeg, wn_ref[128:256, :],
                     preferred_element_type=jnp.float32)
        h = jnp.maximum(h + b_ref[...], 0.0)
        o0[...] = h[:, 0:128]
        o1[...] = h[:, 128:256]
        o2[...] = h[:, 256:384]
        o3[...] = h[:, 384:512]

    grid = (NP // _RB,)
    slab = jax.ShapeDtypeStruct((NP, 128), jnp.float32)
    return pl.pallas_call(
        body,
        grid=grid,
        in_specs=[
            pl.BlockSpec((_RB, 256), lambda i: (i, 0)),
            pl.BlockSpec((2, _RB, 128), lambda i: (0, i, 0)),
            pl.BlockSpec((2, _RB, 128), lambda i: (0, i, 0)),
            pl.BlockSpec((256, 512), lambda i: (0, 0)),
            pl.BlockSpec((256, 512), lambda i: (0, 0)),
            pl.BlockSpec((1, 512), lambda i: (0, 0)),
        ],
        out_specs=[pl.BlockSpec((_RB, 128), lambda i: (i, 0))] * 4,
        out_shape=[slab, slab, slab, slab],
    )(x, agg, deg, Ws1, Wn1, b1)


def _tc2(h_slabs, agg2, deg, Ws2, Wn2, b2, Wp, bp):
    def body(h0, h1, h2s, h3, agg_ref, deg_ref, ws_ref, wn_ref, b_ref,
             wp_ref, bp_ref, s_out, t_out):
        deg = deg_ref[0, :, 0:1] + deg_ref[1, :, 0:1]
        rdeg = 1.0 / jnp.maximum(deg, 1.0)
        hs = (h0, h1, h2s, h3)
        h = b_ref[...] + jnp.zeros((_RB, 512), jnp.float32)
        for ccc in range(4):
            h += jnp.dot(hs[ccc][...], ws_ref[pl.ds(ccc * 128, 128), :],
                         preferred_element_type=jnp.float32)
            h += jnp.dot(agg_ref[ccc] * rdeg, wn_ref[pl.ds(ccc * 128, 128), :],
                         preferred_element_type=jnp.float32)
        s_out[...] = jnp.dot(h, wp_ref[0:512, :],
                             preferred_element_type=jnp.float32)
        t_out[...] = jnp.dot(h, wp_ref[512:1024, :],
                             preferred_element_type=jnp.float32) + bp_ref[...]

    grid = (NP // _RB,)
    out = jax.ShapeDtypeStruct((NP, 256), jnp.float32)
    slab_spec = pl.BlockSpec((_RB, 128), lambda i: (i, 0))
    return pl.pallas_call(
        body,
        grid=grid,
        in_specs=[
            slab_spec, slab_spec, slab_spec, slab_spec,
            pl.BlockSpec((4, _RB, 128), lambda i: (0, i, 0)),
            pl.BlockSpec((2, _RB, 128), lambda i: (0, i, 0)),
            pl.BlockSpec((512, 512), lambda i: (0, 0)),
            pl.BlockSpec((512, 512), lambda i: (0, 0)),
            pl.BlockSpec((1, 512), lambda i: (0, 0)),
            pl.BlockSpec((1024, 256), lambda i: (0, 0)),
            pl.BlockSpec((1, 256), lambda i: (0, 0)),
        ],
        out_specs=[pl.BlockSpec((_RB, 256), lambda i: (i, 0))] * 2,
        out_shape=[out, out],
    )(*h_slabs, agg2, deg, Ws2, Wn2, b2, Wp, bp)


def kernel(x, edge_index, W_self1, W_neigh1, b1, W_self2, W_neigh2, b2,
           Wp, bp):
    src = edge_index[0].astype(jnp.int32)
    dst = edge_index[1].astype(jnp.int32)
    x0 = x[:, 0:128]
    x1 = x[:, 128:256]
    xp = jnp.pad(x, ((0, NP - N), (0, 0)))

    deg = _deg(dst)
    agg1 = _seg_a(x0, x1, src, dst)
    h_slabs = _tc1(xp, agg1, deg, W_self1, W_neigh1, b1.reshape(1, -1))
    agg2 = _seg_b(*h_slabs, src, dst)
    s_tbl, t_tbl = _tc2(h_slabs, agg2, deg, W_self2, W_neigh2,
                        b2.reshape(1, -1), Wp, bp.reshape(1, -1))
    return _score(s_tbl, t_tbl, src, dst)


# speedup vs baseline: 1.0574x; 1.0574x over previous
import functools

import jax
import jax.numpy as jnp
from jax import lax
from jax.experimental import pallas as pl
from jax.experimental.pallas import tpu as pltpu
from jax.experimental.pallas import tpu_sc as plsc

N = 10000
NP = 10240
E = 160000
NC, NS, LANES = 2, 16, 16
ROWS_PER_TILE = NP // NS
EDGES_PER_TILE_AB = E // NS
EDGES_PER_TILE_C = E // (NC * NS)
K = 80
KE = 128
NF_E = EDGES_PER_TILE_AB // KE
REM_E = EDGES_PER_TILE_AB - NF_E * KE
ZROWS = 64

_mesh = functools.partial(
    plsc.VectorSubcoreMesh, core_axis_name="c", subcore_axis_name="s")


def _zero_fill(buf, ncols):
    def row(i, _):
        for j in range(ncols // LANES):
            buf[i, pl.ds(j * LANES, LANES)] = jnp.zeros((LANES,), jnp.float32)
        return 0
    lax.fori_loop(0, ZROWS, row, 0)


def _zero_shared(zbuf, shared, row0, chunk_idx=None):
    for b in range(ROWS_PER_TILE // ZROWS):
        dst = shared.at[pl.ds(row0 + b * ZROWS, ZROWS)]
        pltpu.sync_copy(zbuf, dst)


def _edge_loop(table, src_hbm, dst_hbm, acc_sh, sidx, didx, rows, gsem,
               ssem, sidx_t, didx_t, tile_base):
    def load_idx(g, b):
        base = tile_base + g * KE
        pltpu.sync_copy(src_hbm.at[pl.ds(base, KE)], sidx[b])
        pltpu.sync_copy(dst_hbm.at[pl.ds(base, KE)], didx[b])

    def gather_start(b):
        pltpu.async_copy(table.at[sidx[b]], rows[b], gsem[b])

    def gather_wait(b):
        pltpu.make_async_copy(table.at[sidx[b]], rows[b], gsem[b]).wait()

    def scatter_start(b):
        pltpu.async_copy(rows[b], acc_sh.at[didx[b]], ssem[b], add=True)

    def scatter_wait(b):
        pltpu.make_async_copy(rows[b], acc_sh.at[didx[b]], ssem[b]).wait()

    def body(g, b):
        scatter_wait(b)
        load_idx(g, b)
        gather_start(b)
        gather_wait(1 - b)
        scatter_start(1 - b)

    load_idx(0, 0)
    gather_start(0)
    load_idx(1, 1)
    gather_start(1)
    gather_wait(0)
    scatter_start(0)

    @pl.loop(2, NF_E, step=2)
    def _(g0):
        body(g0, 0)
        body(g0 + 1, 1)

    base_t = tile_base + NF_E * KE
    rows_t = rows[0].at[pl.ds(0, REM_E)]
    scatter_wait(0)
    pltpu.sync_copy(src_hbm.at[pl.ds(base_t, REM_E)], sidx_t)
    pltpu.sync_copy(dst_hbm.at[pl.ds(base_t, REM_E)], didx_t)
    pltpu.async_copy(table.at[sidx_t], rows_t, gsem[0])
    gather_wait(1)
    scatter_start(1)
    pltpu.make_async_copy(table.at[sidx_t], rows_t, gsem[0]).wait()
    pltpu.async_copy(rows_t, acc_sh.at[didx_t], ssem[0], add=True)
    scatter_wait(1)
    pltpu.make_async_copy(rows_t, acc_sh.at[didx_t], ssem[0]).wait()


KD = 128
NF_D = EDGES_PER_TILE_C // KD
REM_D = EDGES_PER_TILE_C - NF_D * KD


def _deg_body(dst_hbm, deg_out, deg_sh, ones_v, didx0, didx1,
              ssem0, ssem1, didx_t):
    c = lax.axis_index("c")
    s = lax.axis_index("s")
    row0 = s * ROWS_PER_TILE
    wid = s * NC + c
    base0 = wid * EDGES_PER_TILE_C
    didx = (didx0, didx1)
    ssem = (ssem0, ssem1)

    _zero_fill(ones_v, 128)
    _zero_shared(ones_v.at[pl.ds(0, ZROWS)], deg_sh, row0)

    plsc.subcore_barrier()

    def orow(i, _):
        for j in range(128 // LANES):
            ones_v[i, pl.ds(j * LANES, LANES)] = jnp.ones((LANES,), jnp.float32)
        return 0
    lax.fori_loop(0, KD, orow, 0)

    def load_idx(g, b):
        pltpu.sync_copy(dst_hbm.at[pl.ds(base0 + g * KD, KD)], didx[b])

    def scatter_start(b):
        pltpu.async_copy(ones_v, deg_sh.at[didx[b]], ssem[b], add=True)

    def scatter_wait(b):
        pltpu.make_async_copy(ones_v, deg_sh.at[didx[b]], ssem[b]).wait()

    def body(g, b):
        scatter_wait(b)
        load_idx(g, b)
        scatter_start(b)

    load_idx(0, 0)
    scatter_start(0)
    load_idx(1, 1)
    scatter_start(1)

    @pl.loop(2, NF_D - 1, step=2)
    def _(g0):
        body(g0, 0)
        body(g0 + 1, 1)

    body(NF_D - 1, 0)

    ones_t = ones_v.at[pl.ds(0, REM_D)]
    scatter_wait(1)
    pltpu.sync_copy(dst_hbm.at[pl.ds(base0 + NF_D * KD, REM_D)], didx_t)
    pltpu.async_copy(ones_t, deg_sh.at[didx_t], ssem[1], add=True)
    scatter_wait(0)
    pltpu.make_async_copy(ones_t, deg_sh.at[didx_t], ssem[1]).wait()

    plsc.subcore_barrier()

    my_rows = pl.ds(row0, ROWS_PER_TILE)

    @pl.when(c == 0)
    def _():
        pltpu.sync_copy(deg_sh.at[my_rows], deg_out.at[0, my_rows])

    @pl.when(c == 1)
    def _():
        pltpu.sync_copy(deg_sh.at[my_rows], deg_out.at[1, my_rows])


def _deg(dst):
    f = pl.kernel(
        _deg_body,
        out_type=jax.ShapeDtypeStruct((2, NP, 128), jnp.float32),
        mesh=_mesh(),
        scratch_types=[
            pltpu.VMEM_SHARED((NP, 128), jnp.float32),
            pltpu.VMEM((KD, 128), jnp.float32),
            pltpu.VMEM((KD,), jnp.int32),
            pltpu.VMEM((KD,), jnp.int32),
            pltpu.SemaphoreType.DMA,
            pltpu.SemaphoreType.DMA,
            pltpu.VMEM((REM_D,), jnp.int32),
        ],
    )
    return f(dst)


def _seg_a_body(x0, x1, src_hbm, dst_hbm, agg_out,
                acc_sh, zbuf, sidx0, sidx1, didx0, didx1, rows0, rows1,
                gsem0, gsem1, ssem0, ssem1, sidx_t, didx_t):
    c = lax.axis_index("c")
    s = lax.axis_index("s")
    row0 = s * ROWS_PER_TILE
    tile_base = s * EDGES_PER_TILE_AB
    sidx = (sidx0, sidx1)
    didx = (didx0, didx1)
    rows = (rows0, rows1)
    gsem = (gsem0, gsem1)
    ssem = (ssem0, ssem1)

    _zero_fill(zbuf, 128)
    _zero_shared(zbuf, acc_sh, row0)

    plsc.subcore_barrier()

    @pl.when(c == 0)
    def _():
        _edge_loop(x0, src_hbm, dst_hbm, acc_sh, sidx, didx, rows, gsem,
                   ssem, sidx_t, didx_t, tile_base)

    @pl.when(c == 1)
    def _():
        _edge_loop(x1, src_hbm, dst_hbm, acc_sh, sidx, didx, rows, gsem,
                   ssem, sidx_t, didx_t, tile_base)

    plsc.subcore_barrier()

    my_rows = pl.ds(row0, ROWS_PER_TILE)

    @pl.when(c == 0)
    def _():
        pltpu.sync_copy(acc_sh.at[my_rows], agg_out.at[0, my_rows])

    @pl.when(c == 1)
    def _():
        pltpu.sync_copy(acc_sh.at[my_rows], agg_out.at[1, my_rows])


def _seg_a(x0, x1, src, dst):
    f = pl.kernel(
        _seg_a_body,
        out_type=jax.ShapeDtypeStruct((2, NP, 128), jnp.float32),
        mesh=_mesh(),
        scratch_types=[
            pltpu.VMEM_SHARED((NP, 128), jnp.float32),
            pltpu.VMEM((ZROWS, 128), jnp.float32),
            pltpu.VMEM((KE,), jnp.int32),
            pltpu.VMEM((KE,), jnp.int32),
            pltpu.VMEM((KE,), jnp.int32),
            pltpu.VMEM((KE,), jnp.int32),
            pltpu.VMEM((KE, 128), jnp.float32),
            pltpu.VMEM((KE, 128), jnp.float32),
            pltpu.SemaphoreType.DMA,
            pltpu.SemaphoreType.DMA,
            pltpu.SemaphoreType.DMA,
            pltpu.SemaphoreType.DMA,
            pltpu.VMEM((REM_E,), jnp.int32),
            pltpu.VMEM((REM_E,), jnp.int32),
        ],
    )
    return f(x0, x1, src, dst)


def _seg_b_body(h0, h1, h2, h3, src_hbm, dst_hbm, agg_out,
                acc_sh, zbuf, sidx0, sidx1, didx0, didx1, rows0, rows1,
                gsem0, gsem1, ssem0, ssem1, sidx_t, didx_t):
    c = lax.axis_index("c")
    s = lax.axis_index("s")
    row0 = s * ROWS_PER_TILE
    tile_base = s * EDGES_PER_TILE_AB
    my_rows = pl.ds(row0, ROWS_PER_TILE)
    sidx = (sidx0, sidx1)
    didx = (didx0, didx1)
    rows = (rows0, rows1)
    gsem = (gsem0, gsem1)
    ssem = (ssem0, ssem1)

    _zero_fill(zbuf, 128)

    def do_chunk(table, out_idx):
        _zero_shared(zbuf, acc_sh, row0)
        plsc.subcore_barrier()
        _edge_loop(table, src_hbm, dst_hbm, acc_sh, sidx, didx, rows, gsem,
                   ssem, sidx_t, didx_t, tile_base)
        plsc.subcore_barrier()
        pltpu.sync_copy(acc_sh.at[my_rows], agg_out.at[out_idx, my_rows])

    @pl.when(c == 0)
    def _():
        do_chunk(h0, 0)
        do_chunk(h1, 1)

    @pl.when(c == 1)
    def _():
        do_chunk(h2, 2)
        do_chunk(h3, 3)


def _seg_b(h0, h1, h2, h3, src, dst):
    f = pl.kernel(
        _seg_b_body,
        out_type=jax.ShapeDtypeStruct((4, NP, 128), jnp.float32),
        mesh=_mesh(),
        scratch_types=[
            pltpu.VMEM_SHARED((NP, 128), jnp.float32),
            pltpu.VMEM((ZROWS, 128), jnp.float32),
            pltpu.VMEM((KE,), jnp.int32),
            pltpu.VMEM((KE,), jnp.int32),
            pltpu.VMEM((KE,), jnp.int32),
            pltpu.VMEM((KE,), jnp.int32),
            pltpu.VMEM((KE, 128), jnp.float32),
            pltpu.VMEM((KE, 128), jnp.float32),
            pltpu.SemaphoreType.DMA,
            pltpu.SemaphoreType.DMA,
            pltpu.SemaphoreType.DMA,
            pltpu.SemaphoreType.DMA,
            pltpu.VMEM((REM_E,), jnp.int32),
            pltpu.VMEM((REM_E,), jnp.int32),
        ],
    )
    return f(h0, h1, h2, h3, src, dst)


def _score_body(s_tbl, t_tbl, src_hbm, dst_hbm, out,
                sidx_all, didx_all, ba0, ba1, bb0, bb1,
                gsa0, gsa1, gsb0, gsb1, wsem0, wsem1):
    c = lax.axis_index("c")
    s = lax.axis_index("s")
    wid = s * NC + c
    base0 = wid * EDGES_PER_TILE_C
    ba = (ba0, ba1)
    bb = (bb0, bb1)
    gsa = (gsa0, gsa1)
    gsb = (gsb0, gsb1)
    wsem = (wsem0, wsem1)

    NF = EDGES_PER_TILE_C // K
    REM = EDGES_PER_TILE_C - NF * K

    pltpu.sync_copy(src_hbm.at[pl.ds(base0, EDGES_PER_TILE_C)], sidx_all)
    pltpu.sync_copy(dst_hbm.at[pl.ds(base0, EDGES_PER_TILE_C)], didx_all)

    def load_and_gather(g, b, n):
        pltpu.async_copy(s_tbl.at[sidx_all.at[pl.ds(g * K, n)]],
                         ba[b].at[pl.ds(0, n)], gsa[b])
        pltpu.async_copy(t_tbl.at[didx_all.at[pl.ds(g * K, n)]],
                         bb[b].at[pl.ds(0, n)], gsb[b])

    def process(g, b, n):
        pltpu.make_async_copy(s_tbl.at[sidx_all.at[pl.ds(g * K, n)]],
                              ba[b].at[pl.ds(0, n)], gsa[b]).wait()
        pltpu.make_async_copy(t_tbl.at[didx_all.at[pl.ds(g * K, n)]],
                              bb[b].at[pl.ds(0, n)], gsb[b]).wait()

        @plsc.parallel_loop(0, n, unroll=4)
        def _(i):
            for j in range(256 // LANES):
                sl = pl.ds(j * LANES, LANES)
                plsc.addupdate(ba[b].at[i, sl], bb[b][i, sl])
        pltpu.async_copy(ba[b].at[pl.ds(0, n)],
                         out.at[pl.ds(base0 + g * K, n)], wsem[b])

    def write_wait(g, b, n):
        pltpu.make_async_copy(ba[b].at[pl.ds(0, n)],
                              out.at[pl.ds(base0 + g * K, n)], wsem[b]).wait()

    def body(g, b):
        write_wait(g - 2, b, K)
        load_and_gather(g, b, K)
        process(g - 1, 1 - b, K)

    load_and_gather(0, 0, K)
    load_and_gather(1, 1, K)
    process(0, 0, K)

    @pl.loop(2, NF, step=2)
    def _(g0):
        body(g0, 0)
        body(g0 + 1, 1)

    write_wait(NF - 2, 0, K)
    load_and_gather(NF, 0, REM)
    process(NF - 1, 1, K)
    write_wait(NF - 1, 1, K)
    process(NF, 0, REM)
    write_wait(NF, 0, REM)


def _score(s_tbl, t_tbl, src, dst):
    f = pl.kernel(
        _score_body,
        out_type=jax.ShapeDtypeStruct((E, 256), jnp.float32),
        mesh=_mesh(),
        scratch_types=[
            pltpu.VMEM((EDGES_PER_TILE_C,), jnp.int32),
            pltpu.VMEM((EDGES_PER_TILE_C,), jnp.int32),
            pltpu.VMEM((K, 256), jnp.float32),
            pltpu.VMEM((K, 256), jnp.float32),
            pltpu.VMEM((K, 256), jnp.float32),
            pltpu.VMEM((K, 256), jnp.float32),
            pltpu.SemaphoreType.DMA,
            pltpu.SemaphoreType.DMA,
            pltpu.SemaphoreType.DMA,
            pltpu.SemaphoreType.DMA,
            pltpu.SemaphoreType.DMA,
            pltpu.SemaphoreType.DMA,
        ],
    )
    return f(s_tbl, t_tbl, src, dst)


_RB = 2048


def _tc1(x, agg, deg, Ws1, Wn1, b1):
    def body(x_ref, agg_ref, deg_ref, ws_ref, wn_ref, b_ref, o0, o1, o2, o3):
        deg = deg_ref[0, :, 0:1] + deg_ref[1, :, 0:1]
        rdeg = 1.0 / jnp.maximum(deg, 1.0)
        h = jnp.dot(x_ref[...], ws_ref[...],
                    preferred_element_type=jnp.float32)
        h += jnp.dot(agg_ref[0] * rdeg, wn_ref[0:128, :],
                     preferred_element_type=jnp.float32)
        h += jnp.dot(agg_ref[1] * rdeg, wn_ref[128:256, :],
                     preferred_element_type=jnp.float32)
        h = jnp.maximum(h + b_ref[...], 0.0)
        o0[...] = h[:, 0:128]
        o1[...] = h[:, 128:256]
        o2[...] = h[:, 256:384]
        o3[...] = h[:, 384:512]

    grid = (NP // _RB,)
    slab = jax.ShapeDtypeStruct((NP, 128), jnp.float32)
    return pl.pallas_call(
        body,
        grid=grid,
        in_specs=[
            pl.BlockSpec((_RB, 256), lambda i: (i, 0)),
            pl.BlockSpec((2, _RB, 128), lambda i: (0, i, 0)),
            pl.BlockSpec((2, _RB, 128), lambda i: (0, i, 0)),
            pl.BlockSpec((256, 512), lambda i: (0, 0)),
            pl.BlockSpec((256, 512), lambda i: (0, 0)),
            pl.BlockSpec((1, 512), lambda i: (0, 0)),
        ],
        out_specs=[pl.BlockSpec((_RB, 128), lambda i: (i, 0))] * 4,
        out_shape=[slab, slab, slab, slab],
    )(x, agg, deg, Ws1, Wn1, b1)


def _tc2(h_slabs, agg2, deg, Ws2, Wn2, b2, Wp, bp):
    def body(h0, h1, h2s, h3, agg_ref, deg_ref, ws_ref, wn_ref, b_ref,
             wp_ref, bp_ref, s_out, t_out):
        deg = deg_ref[0, :, 0:1] + deg_ref[1, :, 0:1]
        rdeg = 1.0 / jnp.maximum(deg, 1.0)
        hs = (h0, h1, h2s, h3)
        h = b_ref[...] + jnp.zeros((_RB, 512), jnp.float32)
        for ccc in range(4):
            h += jnp.dot(hs[ccc][...], ws_ref[pl.ds(ccc * 128, 128), :],
                         preferred_element_type=jnp.float32)
            h += jnp.dot(agg_ref[ccc] * rdeg, wn_ref[pl.ds(ccc * 128, 128), :],
                         preferred_element_type=jnp.float32)
        s_out[...] = jnp.dot(h, wp_ref[0:512, :],
                             preferred_element_type=jnp.float32)
        t_out[...] = jnp.dot(h, wp_ref[512:1024, :],
                             preferred_element_type=jnp.float32) + bp_ref[...]

    grid = (NP // _RB,)
    out = jax.ShapeDtypeStruct((NP, 256), jnp.float32)
    slab_spec = pl.BlockSpec((_RB, 128), lambda i: (i, 0))
    return pl.pallas_call(
        body,
        grid=grid,
        in_specs=[
            slab_spec, slab_spec, slab_spec, slab_spec,
            pl.BlockSpec((4, _RB, 128), lambda i: (0, i, 0)),
            pl.BlockSpec((2, _RB, 128), lambda i: (0, i, 0)),
            pl.BlockSpec((512, 512), lambda i: (0, 0)),
            pl.BlockSpec((512, 512), lambda i: (0, 0)),
            pl.BlockSpec((1, 512), lambda i: (0, 0)),
            pl.BlockSpec((1024, 256), lambda i: (0, 0)),
            pl.BlockSpec((1, 256), lambda i: (0, 0)),
        ],
        out_specs=[pl.BlockSpec((_RB, 256), lambda i: (i, 0))] * 2,
        out_shape=[out, out],
    )(*h_slabs, agg2, deg, Ws2, Wn2, b2, Wp, bp)


def kernel(x, edge_index, W_self1, W_neigh1, b1, W_self2, W_neigh2, b2,
           Wp, bp):
    src = edge_index[0].astype(jnp.int32)
    dst = edge_index[1].astype(jnp.int32)
    x0 = x[:, 0:128]
    x1 = x[:, 128:256]
    xp = jnp.pad(x, ((0, NP - N), (0, 0)))

    deg = _deg(dst)
    agg1 = _seg_a(x0, x1, src, dst)
    h_slabs = _tc1(xp, agg1, deg, W_self1, W_neigh1, b1.reshape(1, -1))
    agg2 = _seg_b(*h_slabs, src, dst)
    s_tbl, t_tbl = _tc2(h_slabs, agg2, deg, W_self2, W_neigh2,
                        b2.reshape(1, -1), Wp, bp.reshape(1, -1))
    return _score(s_tbl, t_tbl, src, dst)


# speedup vs baseline: 1.1837x; 1.1194x over previous
import functools

import jax
import jax.numpy as jnp
from jax import lax
from jax.experimental import pallas as pl
from jax.experimental.pallas import tpu as pltpu
from jax.experimental.pallas import tpu_sc as plsc

N = 10000
NP = 10240
E = 160000
NC, NS, LANES = 2, 16, 16
ROWS_PER_TILE = NP // NS
EDGES_PER_TILE_AB = E // NS
EDGES_PER_TILE_C = E // (NC * NS)
K = 80
KE = 128
NF_E = EDGES_PER_TILE_AB // KE
REM_E = EDGES_PER_TILE_AB - NF_E * KE
ZROWS = 32

_mesh = functools.partial(
    plsc.VectorSubcoreMesh, core_axis_name="c", subcore_axis_name="s")


def _zero_fill(buf, ncols):
    def row(i, _):
        for j in range(ncols // LANES):
            buf[i, pl.ds(j * LANES, LANES)] = jnp.zeros((LANES,), jnp.float32)
        return 0
    lax.fori_loop(0, ZROWS, row, 0)


def _zero_shared(zbuf, shared, row0, chunk_idx=None):
    for b in range(ROWS_PER_TILE // ZROWS):
        dst = shared.at[pl.ds(row0 + b * ZROWS, ZROWS)]
        pltpu.sync_copy(zbuf, dst)


def _edge_loop(table, dst_hbm, acc_sh, sidx_all, didx, rows, gsem,
               ssem, didx_t, tile_base):
    def load_didx(g, b):
        base = tile_base + g * KE
        pltpu.sync_copy(dst_hbm.at[pl.ds(base, KE)], didx[b])

    def gather_start(g, b):
        pltpu.async_copy(table.at[sidx_all.at[pl.ds(g * KE, KE)]],
                         rows[b], gsem[b])

    def gather_wait(g, b):
        pltpu.make_async_copy(table.at[sidx_all.at[pl.ds(g * KE, KE)]],
                              rows[b], gsem[b]).wait()

    def scatter_start(b):
        pltpu.async_copy(rows[b], acc_sh.at[didx[b]], ssem[b], add=True)

    def scatter_wait(b):
        pltpu.make_async_copy(rows[b], acc_sh.at[didx[b]], ssem[b]).wait()

    def body(g, b):
        scatter_wait(b)
        load_didx(g, b)
        gather_start(g, b)
        gather_wait(g - 1, 1 - b)
        scatter_start(1 - b)

    load_didx(0, 0)
    gather_start(0, 0)
    load_didx(1, 1)
    gather_start(1, 1)
    gather_wait(0, 0)
    scatter_start(0)

    @pl.loop(2, NF_E, step=2)
    def _(g0):
        body(g0, 0)
        body(g0 + 1, 1)

    base_t = tile_base + NF_E * KE
    sidx_t = sidx_all.at[pl.ds(NF_E * KE, REM_E)]
    rows_t = rows[0].at[pl.ds(0, REM_E)]
    scatter_wait(0)
    pltpu.sync_copy(dst_hbm.at[pl.ds(base_t, REM_E)], didx_t)
    pltpu.async_copy(table.at[sidx_t], rows_t, gsem[0])
    gather_wait(NF_E - 1, 1)
    scatter_start(1)
    pltpu.make_async_copy(table.at[sidx_t], rows_t, gsem[0]).wait()
    pltpu.async_copy(rows_t, acc_sh.at[didx_t], ssem[0], add=True)
    scatter_wait(1)
    pltpu.make_async_copy(rows_t, acc_sh.at[didx_t], ssem[0]).wait()


KD = 128
NF_D = EDGES_PER_TILE_C // KD
REM_D = EDGES_PER_TILE_C - NF_D * KD


def _deg_body(dst_hbm, deg_out, deg_sh, ones_v, didx0, didx1,
              ssem0, ssem1, didx_t):
    c = lax.axis_index("c")
    s = lax.axis_index("s")
    row0 = s * ROWS_PER_TILE
    wid = s * NC + c
    base0 = wid * EDGES_PER_TILE_C
    didx = (didx0, didx1)
    ssem = (ssem0, ssem1)

    _zero_fill(ones_v, 128)
    _zero_shared(ones_v.at[pl.ds(0, ZROWS)], deg_sh, row0)

    plsc.subcore_barrier()

    def orow(i, _):
        for j in range(128 // LANES):
            ones_v[i, pl.ds(j * LANES, LANES)] = jnp.ones((LANES,), jnp.float32)
        return 0
    lax.fori_loop(0, KD, orow, 0)

    def load_idx(g, b):
        pltpu.sync_copy(dst_hbm.at[pl.ds(base0 + g * KD, KD)], didx[b])

    def scatter_start(b):
        pltpu.async_copy(ones_v, deg_sh.at[didx[b]], ssem[b], add=True)

    def scatter_wait(b):
        pltpu.make_async_copy(ones_v, deg_sh.at[didx[b]], ssem[b]).wait()

    def body(g, b):
        scatter_wait(b)
        load_idx(g, b)
        scatter_start(b)

    load_idx(0, 0)
    scatter_start(0)
    load_idx(1, 1)
    scatter_start(1)

    @pl.loop(2, NF_D - 1, step=2)
    def _(g0):
        body(g0, 0)
        body(g0 + 1, 1)

    body(NF_D - 1, 0)

    ones_t = ones_v.at[pl.ds(0, REM_D)]
    scatter_wait(1)
    pltpu.sync_copy(dst_hbm.at[pl.ds(base0 + NF_D * KD, REM_D)], didx_t)
    pltpu.async_copy(ones_t, deg_sh.at[didx_t], ssem[1], add=True)
    scatter_wait(0)
    pltpu.make_async_copy(ones_t, deg_sh.at[didx_t], ssem[1]).wait()

    plsc.subcore_barrier()

    my_rows = pl.ds(row0, ROWS_PER_TILE)

    @pl.when(c == 0)
    def _():
        pltpu.sync_copy(deg_sh.at[my_rows], deg_out.at[0, my_rows])

    @pl.when(c == 1)
    def _():
        pltpu.sync_copy(deg_sh.at[my_rows], deg_out.at[1, my_rows])


def _deg(dst):
    f = pl.kernel(
        _deg_body,
        out_type=jax.ShapeDtypeStruct((2, NP, 128), jnp.float32),
        mesh=_mesh(),
        scratch_types=[
            pltpu.VMEM_SHARED((NP, 128), jnp.float32),
            pltpu.VMEM((KD, 128), jnp.float32),
            pltpu.VMEM((KD,), jnp.int32),
            pltpu.VMEM((KD,), jnp.int32),
            pltpu.SemaphoreType.DMA,
            pltpu.SemaphoreType.DMA,
            pltpu.VMEM((REM_D,), jnp.int32),
        ],
    )
    return f(dst)


def _seg_a_body(x0, x1, src_hbm, dst_hbm, agg_out,
                acc_sh, zbuf, sidx_all, didx0, didx1, rows0, rows1,
                gsem0, gsem1, ssem0, ssem1, didx_t):
    c = lax.axis_index("c")
    s = lax.axis_index("s")
    row0 = s * ROWS_PER_TILE
    tile_base = s * EDGES_PER_TILE_AB
    didx = (didx0, didx1)
    rows = (rows0, rows1)
    gsem = (gsem0, gsem1)
    ssem = (ssem0, ssem1)

    pltpu.sync_copy(src_hbm.at[pl.ds(tile_base, EDGES_PER_TILE_AB)],
                    sidx_all)
    _zero_fill(zbuf, 128)
    _zero_shared(zbuf, acc_sh, row0)

    plsc.subcore_barrier()

    @pl.when(c == 0)
    def _():
        _edge_loop(x0, dst_hbm, acc_sh, sidx_all, didx, rows, gsem,
                   ssem, didx_t, tile_base)

    @pl.when(c == 1)
    def _():
        _edge_loop(x1, dst_hbm, acc_sh, sidx_all, didx, rows, gsem,
                   ssem, didx_t, tile_base)

    plsc.subcore_barrier()

    my_rows = pl.ds(row0, ROWS_PER_TILE)

    @pl.when(c == 0)
    def _():
        pltpu.sync_copy(acc_sh.at[my_rows], agg_out.at[0, my_rows])

    @pl.when(c == 1)
    def _():
        pltpu.sync_copy(acc_sh.at[my_rows], agg_out.at[1, my_rows])


def _seg_a(x0, x1, src, dst):
    f = pl.kernel(
        _seg_a_body,
        out_type=jax.ShapeDtypeStruct((2, NP, 128), jnp.float32),
        mesh=_mesh(),
        scratch_types=[
            pltpu.VMEM_SHARED((NP, 128), jnp.float32),
            pltpu.VMEM((ZROWS, 128), jnp.float32),
            pltpu.VMEM((EDGES_PER_TILE_AB,), jnp.int32),
            pltpu.VMEM((KE,), jnp.int32),
            pltpu.VMEM((KE,), jnp.int32),
            pltpu.VMEM((KE, 128), jnp.float32),
            pltpu.VMEM((KE, 128), jnp.float32),
            pltpu.SemaphoreType.DMA,
            pltpu.SemaphoreType.DMA,
            pltpu.SemaphoreType.DMA,
            pltpu.SemaphoreType.DMA,
            pltpu.VMEM((REM_E,), jnp.int32),
        ],
    )
    return f(x0, x1, src, dst)


def _seg_b_body(h0, h1, h2, h3, src_hbm, dst_hbm, agg_out,
                acc_sh, zbuf, sidx_all, didx0, didx1, rows0, rows1,
                gsem0, gsem1, ssem0, ssem1, didx_t):
    c = lax.axis_index("c")
    s = lax.axis_index("s")
    row0 = s * ROWS_PER_TILE
    tile_base = s * EDGES_PER_TILE_AB
    my_rows = pl.ds(row0, ROWS_PER_TILE)
    didx = (didx0, didx1)
    rows = (rows0, rows1)
    gsem = (gsem0, gsem1)
    ssem = (ssem0, ssem1)

    pltpu.sync_copy(src_hbm.at[pl.ds(tile_base, EDGES_PER_TILE_AB)],
                    sidx_all)
    _zero_fill(zbuf, 128)

    def do_chunk(table, out_idx):
        _zero_shared(zbuf, acc_sh, row0)
        plsc.subcore_barrier()
        _edge_loop(table, dst_hbm, acc_sh, sidx_all, didx, rows, gsem,
                   ssem, didx_t, tile_base)
        plsc.subcore_barrier()
        pltpu.sync_copy(acc_sh.at[my_rows], agg_out.at[out_idx, my_rows])

    @pl.when(c == 0)
    def _():
        do_chunk(h0, 0)
        do_chunk(h1, 1)

    @pl.when(c == 1)
    def _():
        do_chunk(h2, 2)
        do_chunk(h3, 3)


def _seg_b(h0, h1, h2, h3, src, dst):
    f = pl.kernel(
        _seg_b_body,
        out_type=jax.ShapeDtypeStruct((4, NP, 128), jnp.float32),
        mesh=_mesh(),
        scratch_types=[
            pltpu.VMEM_SHARED((NP, 128), jnp.float32),
            pltpu.VMEM((ZROWS, 128), jnp.float32),
            pltpu.VMEM((EDGES_PER_TILE_AB,), jnp.int32),
            pltpu.VMEM((KE,), jnp.int32),
            pltpu.VMEM((KE,), jnp.int32),
            pltpu.VMEM((KE, 128), jnp.float32),
            pltpu.VMEM((KE, 128), jnp.float32),
            pltpu.SemaphoreType.DMA,
            pltpu.SemaphoreType.DMA,
            pltpu.SemaphoreType.DMA,
            pltpu.SemaphoreType.DMA,
            pltpu.VMEM((REM_E,), jnp.int32),
        ],
    )
    return f(h0, h1, h2, h3, src, dst)


def _score_body(s_tbl, t_tbl, src_hbm, dst_hbm, out,
                sidx_all, didx_all, ba0, ba1, bb0, bb1,
                gsa0, gsa1, gsb0, gsb1, wsem0, wsem1):
    c = lax.axis_index("c")
    s = lax.axis_index("s")
    wid = s * NC + c
    base0 = wid * EDGES_PER_TILE_C
    ba = (ba0, ba1)
    bb = (bb0, bb1)
    gsa = (gsa0, gsa1)
    gsb = (gsb0, gsb1)
    wsem = (wsem0, wsem1)

    NF = EDGES_PER_TILE_C // K
    REM = EDGES_PER_TILE_C - NF * K

    pltpu.sync_copy(src_hbm.at[pl.ds(base0, EDGES_PER_TILE_C)], sidx_all)
    pltpu.sync_copy(dst_hbm.at[pl.ds(base0, EDGES_PER_TILE_C)], didx_all)

    def load_and_gather(g, b, n):
        pltpu.async_copy(s_tbl.at[sidx_all.at[pl.ds(g * K, n)]],
                         ba[b].at[pl.ds(0, n)], gsa[b])
        pltpu.async_copy(t_tbl.at[didx_all.at[pl.ds(g * K, n)]],
                         bb[b].at[pl.ds(0, n)], gsb[b])

    def process(g, b, n):
        pltpu.make_async_copy(s_tbl.at[sidx_all.at[pl.ds(g * K, n)]],
                              ba[b].at[pl.ds(0, n)], gsa[b]).wait()
        pltpu.make_async_copy(t_tbl.at[didx_all.at[pl.ds(g * K, n)]],
                              bb[b].at[pl.ds(0, n)], gsb[b]).wait()

        @plsc.parallel_loop(0, n, unroll=4)
        def _(i):
            for j in range(256 // LANES):
                sl = pl.ds(j * LANES, LANES)
                plsc.addupdate(ba[b].at[i, sl], bb[b][i, sl])
        pltpu.async_copy(ba[b].at[pl.ds(0, n)],
                         out.at[pl.ds(base0 + g * K, n)], wsem[b])

    def write_wait(g, b, n):
        pltpu.make_async_copy(ba[b].at[pl.ds(0, n)],
                              out.at[pl.ds(base0 + g * K, n)], wsem[b]).wait()

    def body(g, b):
        write_wait(g - 2, b, K)
        load_and_gather(g, b, K)
        process(g - 1, 1 - b, K)

    load_and_gather(0, 0, K)
    load_and_gather(1, 1, K)
    process(0, 0, K)

    @pl.loop(2, NF, step=2)
    def _(g0):
        body(g0, 0)
        body(g0 + 1, 1)

    write_wait(NF - 2, 0, K)
    load_and_gather(NF, 0, REM)
    process(NF - 1, 1, K)
    write_wait(NF - 1, 1, K)
    process(NF, 0, REM)
    write_wait(NF, 0, REM)


def _score(s_tbl, t_tbl, src, dst):
    f = pl.kernel(
        _score_body,
        out_type=jax.ShapeDtypeStruct((E, 256), jnp.float32),
        mesh=_mesh(),
        scratch_types=[
            pltpu.VMEM((EDGES_PER_TILE_C,), jnp.int32),
            pltpu.VMEM((EDGES_PER_TILE_C,), jnp.int32),
            pltpu.VMEM((K, 256), jnp.float32),
            pltpu.VMEM((K, 256), jnp.float32),
            pltpu.VMEM((K, 256), jnp.float32),
            pltpu.VMEM((K, 256), jnp.float32),
            pltpu.SemaphoreType.DMA,
            pltpu.SemaphoreType.DMA,
            pltpu.SemaphoreType.DMA,
            pltpu.SemaphoreType.DMA,
            pltpu.SemaphoreType.DMA,
            pltpu.SemaphoreType.DMA,
        ],
    )
    return f(s_tbl, t_tbl, src, dst)


_RB = 2048


def _tc1(x, agg, deg, Ws1, Wn1, b1):
    def body(x_ref, agg_ref, deg_ref, ws_ref, wn_ref, b_ref, o0, o1, o2, o3):
        deg = deg_ref[0, :, 0:1] + deg_ref[1, :, 0:1]
        rdeg = 1.0 / jnp.maximum(deg, 1.0)
        h = jnp.dot(x_ref[...], ws_ref[...],
                    preferred_element_type=jnp.float32)
        h += jnp.dot(agg_ref[0] * rdeg, wn_ref[0:128, :],
                     preferred_element_type=jnp.float32)
        h += jnp.dot(agg_ref[1] * rdeg, wn_ref[128:256, :],
                     preferred_element_type=jnp.float32)
        h = jnp.maximum(h + b_ref[...], 0.0)
        o0[...] = h[:, 0:128]
        o1[...] = h[:, 128:256]
        o2[...] = h[:, 256:384]
        o3[...] = h[:, 384:512]

    grid = (NP // _RB,)
    slab = jax.ShapeDtypeStruct((NP, 128), jnp.float32)
    return pl.pallas_call(
        body,
        grid=grid,
        in_specs=[
            pl.BlockSpec((_RB, 256), lambda i: (i, 0)),
            pl.BlockSpec((2, _RB, 128), lambda i: (0, i, 0)),
            pl.BlockSpec((2, _RB, 128), lambda i: (0, i, 0)),
            pl.BlockSpec((256, 512), lambda i: (0, 0)),
            pl.BlockSpec((256, 512), lambda i: (0, 0)),
            pl.BlockSpec((1, 512), lambda i: (0, 0)),
        ],
        out_specs=[pl.BlockSpec((_RB, 128), lambda i: (i, 0))] * 4,
        out_shape=[slab, slab, slab, slab],
    )(x, agg, deg, Ws1, Wn1, b1)


def _tc2(h_slabs, agg2, deg, Ws2, Wn2, b2, Wp, bp):
    def body(h0, h1, h2s, h3, agg_ref, deg_ref, ws_ref, wn_ref, b_ref,
             wp_ref, bp_ref, s_out, t_out):
        deg = deg_ref[0, :, 0:1] + deg_ref[1, :, 0:1]
        rdeg = 1.0 / jnp.maximum(deg, 1.0)
        hs = (h0, h1, h2s, h3)
        h = b_ref[...] + jnp.zeros((_RB, 512), jnp.float32)
        for ccc in range(4):
            h += jnp.dot(hs[ccc][...], ws_ref[pl.ds(ccc * 128, 128), :],
                         preferred_element_type=jnp.float32)
            h += jnp.dot(agg_ref[ccc] * rdeg, wn_ref[pl.ds(ccc * 128, 128), :],
                         preferred_element_type=jnp.float32)
        s_out[...] = jnp.dot(h, wp_ref[0:512, :],
                             preferred_element_type=jnp.float32)
        t_out[...] = jnp.dot(h, wp_ref[512:1024, :],
                             preferred_element_type=jnp.float32) + bp_ref[...]

    grid = (NP // _RB,)
    out = jax.ShapeDtypeStruct((NP, 256), jnp.float32)
    slab_spec = pl.BlockSpec((_RB, 128), lambda i: (i, 0))
    return pl.pallas_call(
        body,
        grid=grid,
        in_specs=[
            slab_spec, slab_spec, slab_spec, slab_spec,
            pl.BlockSpec((4, _RB, 128), lambda i: (0, i, 0)),
            pl.BlockSpec((2, _RB, 128), lambda i: (0, i, 0)),
            pl.BlockSpec((512, 512), lambda i: (0, 0)),
            pl.BlockSpec((512, 512), lambda i: (0, 0)),
            pl.BlockSpec((1, 512), lambda i: (0, 0)),
            pl.BlockSpec((1024, 256), lambda i: (0, 0)),
            pl.BlockSpec((1, 256), lambda i: (0, 0)),
        ],
        out_specs=[pl.BlockSpec((_RB, 256), lambda i: (i, 0))] * 2,
        out_shape=[out, out],
    )(*h_slabs, agg2, deg, Ws2, Wn2, b2, Wp, bp)


def kernel(x, edge_index, W_self1, W_neigh1, b1, W_self2, W_neigh2, b2,
           Wp, bp):
    src = edge_index[0].astype(jnp.int32)
    dst = edge_index[1].astype(jnp.int32)
    x0 = x[:, 0:128]
    x1 = x[:, 128:256]
    xp = jnp.pad(x, ((0, NP - N), (0, 0)))

    deg = _deg(dst)
    agg1 = _seg_a(x0, x1, src, dst)
    h_slabs = _tc1(xp, agg1, deg, W_self1, W_neigh1, b1.reshape(1, -1))
    agg2 = _seg_b(*h_slabs, src, dst)
    s_tbl, t_tbl = _tc2(h_slabs, agg2, deg, W_self2, W_neigh2,
                        b2.reshape(1, -1), Wp, bp.reshape(1, -1))
    return _score(s_tbl, t_tbl, src, dst)
